# baseline (speedup 1.0000x reference)
import jax
import jax.numpy as jnp
from jax import lax
from jax.experimental import pallas as pl
from jax.experimental.pallas import tpu as pltpu
from jax.experimental.pallas import tpu_sc as plsc

N_NODES = 10000
N_PAD = 10240
D = 128
DH = 64
K = 80
NT = 16
ROWS = N_PAD // NT
RSUB = 64
NCH = 256
E_PAD = NT * NCH * K
NIDX = 8
NBUF = 4


def _make_sc_kernel():
    mesh = plsc.VectorSubcoreMesh(core_axis_name="c", subcore_axis_name="s")
    per_tile = NCH * K

    def body(nf_hbm, src_hbm, dst_hbm, cnt_hbm, z_hbm, out_hbm,
             nf_loc, acc_loc, dpl, deg_v, deg_loc, cnt_loc,
             idx_s0, idx_s1, idx_s2, idx_s3, idx_s4, idx_s5, idx_s6, idx_s7,
             idx_d0, idx_d1, idx_d2, idx_d3, idx_d4, idx_d5, idx_d6, idx_d7,
             buf_a0, buf_a1, buf_a2, buf_a3,
             buf_b0, buf_b1, buf_b2, buf_b3,
             sem_i0, sem_i1, sem_i2, sem_i3, sem_i4, sem_i5, sem_i6, sem_i7,
             sem_ga0, sem_ga1, sem_ga2, sem_ga3,
             sem_gb0, sem_gb1, sem_gb2, sem_gb3,
             sem_sa0, sem_sa1, sem_sb0, sem_sb1,
             acc_sh, deg_part):
        c = lax.axis_index("c")
        s = lax.axis_index("s")
        r0 = s * ROWS
        zero16 = jnp.zeros((16,), jnp.float32)
        ones16 = jnp.full((16,), 1.0, jnp.float32)
        idx_s = [idx_s0, idx_s1, idx_s2, idx_s3, idx_s4, idx_s5, idx_s6, idx_s7]
        idx_d = [idx_d0, idx_d1, idx_d2, idx_d3, idx_d4, idx_d5, idx_d6, idx_d7]
        buf_a = [buf_a0, buf_a1, buf_a2, buf_a3]
        buf_b = [buf_b0, buf_b1, buf_b2, buf_b3]
        sem_i = [sem_i0, sem_i1, sem_i2, sem_i3, sem_i4, sem_i5, sem_i6, sem_i7]
        sem_ga = [sem_ga0, sem_ga1, sem_ga2, sem_ga3]
        sem_gb = [sem_gb0, sem_gb1, sem_gb2, sem_gb3]
        sem_sa = [sem_sa0, sem_sa1]
        sem_sb = [sem_sb0, sem_sb1]

        ebase = s * per_tile

        pltpu.sync_copy(z_hbm, acc_loc)
        for cc in range(ROWS // RSUB):
            pltpu.sync_copy(acc_loc, acc_sh.at[pl.ds(r0 + cc * RSUB, RSUB)])

        def _zdeg(i, carry):
            deg_v[pl.ds(i * 16, 16)] = zero16
            return carry
        lax.fori_loop(0, N_PAD // 16, _zdeg, 0)

        plsc.subcore_barrier()

        def stage_idx(k, kk):
            j = kk % NIDX
            off = ebase + k * K
            pltpu.async_copy(src_hbm.at[pl.ds(off, K)], idx_s[j], sem_i[j])
            pltpu.async_copy(dst_hbm.at[pl.ds(off, K)], idx_d[j], sem_i[j])

        def wait_idx(k, kk):
            j = kk % NIDX
            off = ebase + k * K
            pltpu.make_async_copy(src_hbm.at[pl.ds(off, K)], idx_s[j], sem_i[j]).wait()
            pltpu.make_async_copy(dst_hbm.at[pl.ds(off, K)], idx_d[j], sem_i[j]).wait()

        def issue_gathers(kk):
            j = kk % NIDX
            b = kk % NBUF
            pltpu.async_copy(nf_hbm.at[c].at[idx_d[j]], buf_b[b], sem_gb[b])
            pltpu.async_copy(nf_hbm.at[c].at[idx_s[j]], buf_a[b], sem_ga[b])

        def wait_gathers(kk):
            j = kk % NIDX
            b = kk % NBUF
            pltpu.make_async_copy(nf_hbm.at[c].at[idx_d[j]], buf_b[b], sem_gb[b]).wait()
            pltpu.make_async_copy(nf_hbm.at[c].at[idx_s[j]], buf_a[b], sem_ga[b]).wait()

        def issue_scatters(kk):
            j = kk % NIDX
            b = kk % NBUF
            pltpu.async_copy(buf_b[b], acc_sh.at[idx_s[j]], sem_sb[kk % 2], add=True)
            pltpu.async_copy(buf_a[b], acc_sh.at[idx_d[j]], sem_sa[kk % 2], add=True)

        def wait_scatters(kk):
            b = kk % NBUF
            pltpu.make_async_copy(buf_b[b], acc_sh.at[idx_s[0]], sem_sb[kk % 2]).wait()
            pltpu.make_async_copy(buf_a[b], acc_sh.at[idx_d[0]], sem_sa[kk % 2]).wait()

        def deg_count(kk):
            j = kk % NIDX
            for jj in range(K // 16):
                plsc.addupdate_scatter(deg_v, [idx_s[j][pl.ds(jj * 16, 16)]], ones16)
                plsc.addupdate_scatter(deg_v, [idx_d[j][pl.ds(jj * 16, 16)]], ones16)

        def chunk_body(k, kk, wait_prev, stage_next, gather_next):
            if wait_prev:
                wait_scatters(kk - 2)
            if stage_next:
                stage_idx(k + 4, kk + 4)
            if gather_next:
                wait_idx(k + 2, kk + 2)
                issue_gathers(kk + 2)
            deg_count(kk)
            wait_gathers(kk)
            issue_scatters(kk)

        for k in range(4):
            stage_idx(k, k)
        for k in range(2):
            wait_idx(k, k)
            issue_gathers(k)
        chunk_body(0, 0, False, True, True)
        chunk_body(1, 1, False, True, True)
        for kk in range(2, 8):
            chunk_body(kk, kk, True, True, True)

        def _iter(i, carry):
            kbase = i * NIDX
            for kk in range(NIDX):
                chunk_body(kbase + kk, kk, True, True, True)
            return carry
        lax.fori_loop(1, NCH // NIDX - 1, _iter, 0)

        for kk in range(NCH - 8, NCH):
            chunk_body(kk, kk % NIDX, True, stage_next=(kk + 4 < NCH),
                       gather_next=(kk + 2 < NCH))
        wait_scatters(NCH - 2)
        wait_scatters(NCH - 1)

        pltpu.sync_copy(deg_v, deg_part.at[s])
        plsc.subcore_barrier()
        for cc in range(ROWS // RSUB):
            pltpu.sync_copy(deg_part.at[:, pl.ds(r0 + cc * RSUB, RSUB)], dpl)

            def _gbody(g, carry):
                a = dpl[0, pl.ds(g * 16, 16)]
                for j in range(1, NT):
                    a = a + dpl[j, pl.ds(g * 16, 16)]
                deg_loc[pl.ds(cc * RSUB + g * 16, 16)] = a
                return carry
            lax.fori_loop(0, RSUB // 16, _gbody, 0)

        pltpu.sync_copy(cnt_hbm.at[pl.ds(r0, ROWS)], cnt_loc)
        for cc in range(ROWS // RSUB):
            rc = r0 + cc * RSUB
            pltpu.sync_copy(nf_hbm.at[c, pl.ds(rc, RSUB), :], nf_loc)
            pltpu.sync_copy(acc_sh.at[pl.ds(rc, RSUB)], acc_loc)

            def _egrp(g, carry):
                dg16 = deg_loc[pl.ds(cc * RSUB + g * 16, 16)]
                sc16 = 0.5 / cnt_loc[pl.ds(cc * RSUB + g * 16, 16)]
                for lane in range(16):
                    r = g * 16 + lane
                    dg = dg16[lane]
                    sc = sc16[lane]
                    for j in range(DH // 16):
                        sl = pl.ds(j * 16, 16)
                        acc_loc[r, sl] = (acc_loc[r, sl] + dg * nf_loc[r, sl]) * sc
                return carry
            lax.fori_loop(0, RSUB // 16, _egrp, 0)
            pltpu.sync_copy(acc_loc, out_hbm.at[c, pl.ds(rc, RSUB), :])

    return pl.kernel(
        body,
        out_type=jax.ShapeDtypeStruct((2, N_PAD, DH), jnp.float32),
        mesh=mesh,
        compiler_params=pltpu.CompilerParams(
            needs_layout_passes=False, use_tc_tiling_on_sc=False),
        scratch_types=(
            [
                pltpu.VMEM((RSUB, DH), jnp.float32),
                pltpu.VMEM((RSUB, DH), jnp.float32),
                pltpu.VMEM((NT, RSUB), jnp.float32),
                pltpu.VMEM((N_PAD,), jnp.float32),
                pltpu.VMEM((ROWS,), jnp.float32),
                pltpu.VMEM((ROWS,), jnp.float32),
            ]
            + [pltpu.VMEM((K,), jnp.int32) for _ in range(2 * NIDX)]
            + [pltpu.VMEM((K, DH), jnp.float32) for _ in range(2 * NBUF)]
            + [pltpu.SemaphoreType.DMA for _ in range(NIDX + 2 * NBUF + 4)]
            + [
                pltpu.VMEM_SHARED((N_PAD, DH), jnp.float32),
                pltpu.VMEM_SHARED((NT, N_PAD), jnp.float32),
            ]
        ),
    )


def kernel(to_concat, node_features, edges, count):
    nf = node_features.astype(jnp.float32)
    src = edges[:, 0].astype(jnp.int32)
    dst = edges[:, 1].astype(jnp.int32)
    n_edges = src.shape[0]
    src = jnp.pad(src, (0, E_PAD - n_edges), constant_values=N_PAD - 1)
    dst = jnp.pad(dst, (0, E_PAD - n_edges), constant_values=N_PAD - 1)
    nf_pad = jnp.pad(nf, ((0, N_PAD - N_NODES), (0, 0)))
    nf_halves = jnp.transpose(nf_pad.reshape(N_PAD, 2, DH), (1, 0, 2))
    cnt = jnp.pad(count[:, 0].astype(jnp.float32), (0, N_PAD - N_NODES),
                  constant_values=1.0)
    zrows = jnp.zeros((RSUB, DH), jnp.float32)
    sm = _make_sc_kernel()(nf_halves, src, dst, cnt, zrows)
    sm = jnp.transpose(sm, (1, 0, 2)).reshape(N_PAD, D)
    return jnp.concatenate([to_concat, sm[:N_NODES]], axis=1)


# speedup vs baseline: 1.2241x; 1.2241x over previous
import jax
import jax.numpy as jnp
from jax import lax
from jax.experimental import pallas as pl
from jax.experimental.pallas import tpu as pltpu
from jax.experimental.pallas import tpu_sc as plsc

N_NODES = 10000
N_PAD = 10240
D = 128
DH = 64
DW = 8
K = 128
NT = 16
ROWS = N_PAD // NT
RSUB = 32
NCH = 160
E_PAD = NT * NCH * K
NIDX = 4
NBUF = 2


def _make_sc_kernel():
    mesh = plsc.VectorSubcoreMesh(core_axis_name="c", subcore_axis_name="s")
    per_tile = NCH * K

    def body(nf_hbm, src_hbm, dst_hbm, cnt_hbm, z_hbm, z8_hbm, ones8_hbm,
             out_hbm,
             nf_loc, acc_loc, cnt_loc, degl, ones8,
             idx_s0, idx_s1, idx_s2, idx_s3,
             idx_d0, idx_d1, idx_d2, idx_d3,
             buf_a0, buf_a1, buf_b0, buf_b1,
             sem_i0, sem_i1, sem_i2, sem_i3,
             sem_ga0, sem_ga1, sem_gb0, sem_gb1,
             sem_sa0, sem_sa1, sem_sb0, sem_sb1,
             sem_da0, sem_da1, sem_db0, sem_db1,
             nf_sh, acc_sh, deg_sh):
        c = lax.axis_index("c")
        s = lax.axis_index("s")
        r0 = s * ROWS
        idx_s = [idx_s0, idx_s1, idx_s2, idx_s3]
        idx_d = [idx_d0, idx_d1, idx_d2, idx_d3]
        buf_a = [buf_a0, buf_a1]
        buf_b = [buf_b0, buf_b1]
        sem_i = [sem_i0, sem_i1, sem_i2, sem_i3]
        sem_ga = [sem_ga0, sem_ga1]
        sem_gb = [sem_gb0, sem_gb1]
        sem_sa = [sem_sa0, sem_sa1]
        sem_sb = [sem_sb0, sem_sb1]
        sem_da = [sem_da0, sem_da1]
        sem_db = [sem_db0, sem_db1]

        ebase = s * per_tile

        pltpu.sync_copy(ones8_hbm, ones8)
        pltpu.sync_copy(z_hbm, acc_loc)
        pltpu.sync_copy(z8_hbm, degl)
        for cc in range(ROWS // RSUB):
            rc = r0 + cc * RSUB
            pltpu.sync_copy(nf_hbm.at[c, pl.ds(rc, RSUB), :], nf_loc)
            pltpu.sync_copy(nf_loc, nf_sh.at[pl.ds(rc, RSUB)])
            pltpu.sync_copy(acc_loc, acc_sh.at[pl.ds(rc, RSUB)])
            pltpu.sync_copy(degl, deg_sh.at[pl.ds(rc, RSUB)])

        plsc.subcore_barrier()

        def stage_idx(k, kk):
            j = kk % NIDX
            off = ebase + k * K
            pltpu.async_copy(src_hbm.at[pl.ds(off, K)], idx_s[j], sem_i[j])
            pltpu.async_copy(dst_hbm.at[pl.ds(off, K)], idx_d[j], sem_i[j])

        def wait_idx(k, kk):
            j = kk % NIDX
            off = ebase + k * K
            pltpu.make_async_copy(src_hbm.at[pl.ds(off, K)], idx_s[j], sem_i[j]).wait()
            pltpu.make_async_copy(dst_hbm.at[pl.ds(off, K)], idx_d[j], sem_i[j]).wait()

        def issue_gathers(kk):
            j = kk % NIDX
            b = kk % NBUF
            pltpu.async_copy(nf_sh.at[idx_d[j]], buf_b[b], sem_gb[b])
            pltpu.async_copy(nf_sh.at[idx_s[j]], buf_a[b], sem_ga[b])

        def wait_gathers(kk):
            j = kk % NIDX
            b = kk % NBUF
            pltpu.make_async_copy(nf_sh.at[idx_d[j]], buf_b[b], sem_gb[b]).wait()
            pltpu.make_async_copy(nf_sh.at[idx_s[j]], buf_a[b], sem_ga[b]).wait()

        def issue_scatters(kk):
            j = kk % NIDX
            b = kk % NBUF
            p = kk % 2
            pltpu.async_copy(buf_b[b], acc_sh.at[idx_s[j]], sem_sb[p], add=True)
            pltpu.async_copy(buf_a[b], acc_sh.at[idx_d[j]], sem_sa[p], add=True)
            pltpu.async_copy(ones8, deg_sh.at[idx_s[j]], sem_db[p], add=True)
            pltpu.async_copy(ones8, deg_sh.at[idx_d[j]], sem_da[p], add=True)

        def wait_scatters(kk):
            b = kk % NBUF
            p = kk % 2
            pltpu.make_async_copy(buf_b[b], acc_sh.at[idx_s[0]], sem_sb[p]).wait()
            pltpu.make_async_copy(buf_a[b], acc_sh.at[idx_d[0]], sem_sa[p]).wait()
            pltpu.make_async_copy(ones8, deg_sh.at[idx_s[0]], sem_db[p]).wait()
            pltpu.make_async_copy(ones8, deg_sh.at[idx_d[0]], sem_da[p]).wait()

        def chunk_body(k, kk, wait_prev, stage_next):
            if wait_prev:
                wait_scatters(kk - 2)
            if stage_next:
                stage_idx(k + 2, kk + 2)
            wait_idx(k, kk)
            issue_gathers(kk)
            wait_gathers(kk)
            issue_scatters(kk)

        stage_idx(0, 0)
        stage_idx(1, 1)
        chunk_body(0, 0, False, True)
        chunk_body(1, 1, False, True)
        chunk_body(2, 2, True, True)
        chunk_body(3, 3, True, True)

        def _iter(i, carry):
            kbase = i * NIDX
            for kk in range(NIDX):
                chunk_body(kbase + kk, kk, True, True)
            return carry
        lax.fori_loop(1, NCH // NIDX - 1, _iter, 0)

        for kk in range(NCH - 4, NCH):
            chunk_body(kk, kk % NIDX, True, stage_next=(kk + 2 < NCH))
        wait_scatters(NCH - 2)
        wait_scatters(NCH - 1)

        plsc.subcore_barrier()

        pltpu.sync_copy(cnt_hbm.at[pl.ds(r0, ROWS)], cnt_loc)
        col0 = jnp.zeros((16,), jnp.int32)
        for cc in range(ROWS // RSUB):
            rc = r0 + cc * RSUB
            pltpu.sync_copy(nf_hbm.at[c, pl.ds(rc, RSUB), :], nf_loc)
            pltpu.sync_copy(acc_sh.at[pl.ds(rc, RSUB)], acc_loc)
            pltpu.sync_copy(deg_sh.at[pl.ds(rc, RSUB)], degl)

            def _egrp(g, carry):
                sc16 = 0.5 / cnt_loc[pl.ds(cc * RSUB + g * 16, 16)]
                rows16 = g * 16 + lax.iota(jnp.int32, 16)
                dg16 = plsc.load_gather(degl, [rows16, col0])
                for lane in range(16):
                    r = g * 16 + lane
                    dg = dg16[lane]
                    sc = sc16[lane]
                    for j in range(DH // 16):
                        sl = pl.ds(j * 16, 16)
                        acc_loc[r, sl] = (acc_loc[r, sl] + dg * nf_loc[r, sl]) * sc
                return carry
            lax.fori_loop(0, RSUB // 16, _egrp, 0)
            pltpu.sync_copy(acc_loc, out_hbm.at[c, pl.ds(rc, RSUB), :])

    return pl.kernel(
        body,
        out_type=jax.ShapeDtypeStruct((2, N_PAD, DH), jnp.float32),
        mesh=mesh,
        compiler_params=pltpu.CompilerParams(
            needs_layout_passes=False, use_tc_tiling_on_sc=False),
        scratch_types=(
            [
                pltpu.VMEM((RSUB, DH), jnp.float32),
                pltpu.VMEM((RSUB, DH), jnp.float32),
                pltpu.VMEM((ROWS,), jnp.float32),
                pltpu.VMEM((RSUB, DW), jnp.float32),
                pltpu.VMEM((K, DW), jnp.float32),
            ]
            + [pltpu.VMEM((K,), jnp.int32) for _ in range(2 * NIDX)]
            + [pltpu.VMEM((K, DH), jnp.float32) for _ in range(2 * NBUF)]
            + [pltpu.SemaphoreType.DMA for _ in range(NIDX + 2 * NBUF + 8)]
            + [
                pltpu.VMEM_SHARED((N_PAD, DH), jnp.float32),
                pltpu.VMEM_SHARED((N_PAD, DH), jnp.float32),
                pltpu.VMEM_SHARED((N_PAD, DW), jnp.float32),
            ]
        ),
    )


def kernel(to_concat, node_features, edges, count):
    nf = node_features.astype(jnp.float32)
    src = edges[:, 0].astype(jnp.int32)
    dst = edges[:, 1].astype(jnp.int32)
    n_edges = src.shape[0]
    src = jnp.pad(src, (0, E_PAD - n_edges), constant_values=N_PAD - 1)
    dst = jnp.pad(dst, (0, E_PAD - n_edges), constant_values=N_PAD - 1)
    nf_pad = jnp.pad(nf, ((0, N_PAD - N_NODES), (0, 0)))
    nf_halves = jnp.transpose(nf_pad.reshape(N_PAD, 2, DH), (1, 0, 2))
    cnt = jnp.pad(count[:, 0].astype(jnp.float32), (0, N_PAD - N_NODES),
                  constant_values=1.0)
    zrows = jnp.zeros((RSUB, DH), jnp.float32)
    z8 = jnp.zeros((RSUB, DW), jnp.float32)
    ones8 = jnp.ones((K, DW), jnp.float32)
    sm = _make_sc_kernel()(nf_halves, src, dst, cnt, zrows, z8, ones8)
    sm = jnp.transpose(sm, (1, 0, 2)).reshape(N_PAD, D)
    return jnp.concatenate([to_concat, sm[:N_NODES]], axis=1)


# speedup vs baseline: 1.3150x; 1.0743x over previous
import jax
import jax.numpy as jnp
from jax import lax
from jax.experimental import pallas as pl
from jax.experimental.pallas import tpu as pltpu
from jax.experimental.pallas import tpu_sc as plsc

N_NODES = 10000
N_PAD = 10240
D = 128
DH = 64
DW = 8
K = 80
NT = 16
ROWS = N_PAD // NT
RSUB = 32
NCH = 256
E_PAD = NT * NCH * K
NIDX = 4
NBUF = 2


def _make_sc_kernel():
    mesh = plsc.VectorSubcoreMesh(core_axis_name="c", subcore_axis_name="s")
    per_tile = NCH * K

    def body(nf_hbm, src_hbm, dst_hbm, cnt_hbm, z_hbm, out_hbm,
             nf_loc, acc_loc, cnt_loc, dpl, deg_v, deg_loc,
             idx_s0, idx_s1, idx_s2, idx_s3,
             idx_d0, idx_d1, idx_d2, idx_d3,
             buf_a0, buf_a1, buf_b0, buf_b1,
             sem_i0, sem_i1, sem_i2, sem_i3,
             sem_ga0, sem_ga1, sem_gb0, sem_gb1,
             sem_sa0, sem_sa1, sem_sb0, sem_sb1,
             nf_sh, acc_sh, deg_part):
        c = lax.axis_index("c")
        s = lax.axis_index("s")
        r0 = s * ROWS
        idx_s = [idx_s0, idx_s1, idx_s2, idx_s3]
        idx_d = [idx_d0, idx_d1, idx_d2, idx_d3]
        buf_a = [buf_a0, buf_a1]
        buf_b = [buf_b0, buf_b1]
        sem_i = [sem_i0, sem_i1, sem_i2, sem_i3]
        sem_ga = [sem_ga0, sem_ga1]
        sem_gb = [sem_gb0, sem_gb1]
        sem_sa = [sem_sa0, sem_sa1]
        sem_sb = [sem_sb0, sem_sb1]
        zero16 = jnp.zeros((16,), jnp.float32)
        ones16 = jnp.full((16,), 1.0, jnp.float32)

        ebase = s * per_tile

        pltpu.sync_copy(z_hbm, acc_loc)
        for cc in range(ROWS // RSUB):
            rc = r0 + cc * RSUB
            pltpu.sync_copy(nf_hbm.at[c, pl.ds(rc, RSUB), :], nf_loc)
            pltpu.sync_copy(nf_loc, nf_sh.at[pl.ds(rc, RSUB)])
            pltpu.sync_copy(acc_loc, acc_sh.at[pl.ds(rc, RSUB)])

        def _zdeg(i, carry):
            deg_v[pl.ds(i * 16, 16)] = zero16
            return carry
        lax.fori_loop(0, N_PAD // 16, _zdeg, 0)

        plsc.subcore_barrier()

        def stage_idx(k, kk):
            j = kk % NIDX
            off = ebase + k * K
            pltpu.async_copy(src_hbm.at[pl.ds(off, K)], idx_s[j], sem_i[j])
            pltpu.async_copy(dst_hbm.at[pl.ds(off, K)], idx_d[j], sem_i[j])

        def wait_idx(k, kk):
            j = kk % NIDX
            off = ebase + k * K
            pltpu.make_async_copy(src_hbm.at[pl.ds(off, K)], idx_s[j], sem_i[j]).wait()
            pltpu.make_async_copy(dst_hbm.at[pl.ds(off, K)], idx_d[j], sem_i[j]).wait()

        def issue_gathers(kk):
            j = kk % NIDX
            b = kk % NBUF
            pltpu.async_copy(nf_sh.at[idx_d[j]], buf_b[b], sem_gb[b])
            pltpu.async_copy(nf_sh.at[idx_s[j]], buf_a[b], sem_ga[b])

        def wait_gathers(kk):
            j = kk % NIDX
            b = kk % NBUF
            pltpu.make_async_copy(nf_sh.at[idx_d[j]], buf_b[b], sem_gb[b]).wait()
            pltpu.make_async_copy(nf_sh.at[idx_s[j]], buf_a[b], sem_ga[b]).wait()

        def issue_scatters(kk):
            j = kk % NIDX
            b = kk % NBUF
            p = kk % 2
            pltpu.async_copy(buf_b[b], acc_sh.at[idx_s[j]], sem_sb[p], add=True)
            pltpu.async_copy(buf_a[b], acc_sh.at[idx_d[j]], sem_sa[p], add=True)

        def wait_scatters(kk):
            b = kk % NBUF
            p = kk % 2
            pltpu.make_async_copy(buf_b[b], acc_sh.at[idx_s[0]], sem_sb[p]).wait()
            pltpu.make_async_copy(buf_a[b], acc_sh.at[idx_d[0]], sem_sa[p]).wait()

        def chunk_body(k, kk, wait_prev, stage_next):
            if wait_prev:
                wait_scatters(kk - 2)
            if stage_next:
                stage_idx(k + 2, kk + 2)
            wait_idx(k, kk)
            j = kk % NIDX
            for jj in range(K // 16):
                plsc.addupdate_scatter(deg_v, [idx_s[j][pl.ds(jj * 16, 16)]], ones16)
                plsc.addupdate_scatter(deg_v, [idx_d[j][pl.ds(jj * 16, 16)]], ones16)
            issue_gathers(kk)
            wait_gathers(kk)
            issue_scatters(kk)

        stage_idx(0, 0)
        stage_idx(1, 1)
        chunk_body(0, 0, False, True)
        chunk_body(1, 1, False, True)
        chunk_body(2, 2, True, True)
        chunk_body(3, 3, True, True)

        def _iter(i, carry):
            kbase = i * NIDX
            for kk in range(NIDX):
                chunk_body(kbase + kk, kk, True, True)
            return carry
        lax.fori_loop(1, NCH // NIDX - 1, _iter, 0)

        for kk in range(NCH - 4, NCH):
            chunk_body(kk, kk % NIDX, True, stage_next=(kk + 2 < NCH))
        wait_scatters(NCH - 2)
        wait_scatters(NCH - 1)

        pltpu.sync_copy(deg_v, deg_part.at[s])
        plsc.subcore_barrier()
        for cc in range(ROWS // RSUB):
            pltpu.sync_copy(deg_part.at[:, pl.ds(r0 + cc * RSUB, RSUB)], dpl)

            def _gbody(g, carry):
                a = dpl[0, pl.ds(g * 16, 16)]
                for j in range(1, NT):
                    a = a + dpl[j, pl.ds(g * 16, 16)]
                deg_loc[pl.ds(cc * RSUB + g * 16, 16)] = a
                return carry
            lax.fori_loop(0, RSUB // 16, _gbody, 0)

        pltpu.sync_copy(cnt_hbm.at[pl.ds(r0, ROWS)], cnt_loc)
        for cc in range(ROWS // RSUB):
            rc = r0 + cc * RSUB
            pltpu.sync_copy(nf_hbm.at[c, pl.ds(rc, RSUB), :], nf_loc)
            pltpu.sync_copy(acc_sh.at[pl.ds(rc, RSUB)], acc_loc)

            def _egrp(g, carry):
                sc16 = 0.5 / cnt_loc[pl.ds(cc * RSUB + g * 16, 16)]
                dg16 = deg_loc[pl.ds(cc * RSUB + g * 16, 16)]
                for lane in range(16):
                    r = g * 16 + lane
                    dg = dg16[lane]
                    sc = sc16[lane]
                    for j in range(DH // 16):
                        sl = pl.ds(j * 16, 16)
                        acc_loc[r, sl] = (acc_loc[r, sl] + dg * nf_loc[r, sl]) * sc
                return carry
            lax.fori_loop(0, RSUB // 16, _egrp, 0)
            pltpu.sync_copy(acc_loc, out_hbm.at[c, pl.ds(rc, RSUB), :])

    return pl.kernel(
        body,
        out_type=jax.ShapeDtypeStruct((2, N_PAD, DH), jnp.float32),
        mesh=mesh,
        compiler_params=pltpu.CompilerParams(
            needs_layout_passes=False, use_tc_tiling_on_sc=False),
        scratch_types=(
            [
                pltpu.VMEM((RSUB, DH), jnp.float32),
                pltpu.VMEM((RSUB, DH), jnp.float32),
                pltpu.VMEM((ROWS,), jnp.float32),
                pltpu.VMEM((NT, RSUB), jnp.float32),
                pltpu.VMEM((N_PAD,), jnp.float32),
                pltpu.VMEM((ROWS,), jnp.float32),
            ]
            + [pltpu.VMEM((K,), jnp.int32) for _ in range(2 * NIDX)]
            + [pltpu.VMEM((K, DH), jnp.float32) for _ in range(2 * NBUF)]
            + [pltpu.SemaphoreType.DMA for _ in range(NIDX + 2 * NBUF + 4)]
            + [
                pltpu.VMEM_SHARED((N_PAD, DH), jnp.float32),
                pltpu.VMEM_SHARED((N_PAD, DH), jnp.float32),
                pltpu.VMEM_SHARED((NT, N_PAD), jnp.float32),
            ]
        ),
    )


def kernel(to_concat, node_features, edges, count):
    nf = node_features.astype(jnp.float32)
    src = edges[:, 0].astype(jnp.int32)
    dst = edges[:, 1].astype(jnp.int32)
    n_edges = src.shape[0]
    src = jnp.pad(src, (0, E_PAD - n_edges), constant_values=N_PAD - 1)
    dst = jnp.pad(dst, (0, E_PAD - n_edges), constant_values=N_PAD - 1)
    nf_pad = jnp.pad(nf, ((0, N_PAD - N_NODES), (0, 0)))
    nf_halves = jnp.transpose(nf_pad.reshape(N_PAD, 2, DH), (1, 0, 2))
    cnt = jnp.pad(count[:, 0].astype(jnp.float32), (0, N_PAD - N_NODES),
                  constant_values=1.0)
    zrows = jnp.zeros((RSUB, DH), jnp.float32)
    sm = _make_sc_kernel()(nf_halves, src, dst, cnt, zrows)
    sm = jnp.transpose(sm, (1, 0, 2)).reshape(N_PAD, D)
    return jnp.concatenate([to_concat, sm[:N_NODES]], axis=1)


# speedup vs baseline: 1.3253x; 1.0078x over previous
import jax
import jax.numpy as jnp
from jax import lax
from jax.experimental import pallas as pl
from jax.experimental.pallas import tpu as pltpu
from jax.experimental.pallas import tpu_sc as plsc

N_NODES = 10000
N_PAD = 10240
D = 128
DH = 64
DW = 8
K = 64
NT = 16
ROWS = N_PAD // NT
RSUB = 64
NCH = 320
E_PAD = NT * NCH * K
NIDX = 4
NBUF = 2


def _make_sc_kernel():
    mesh = plsc.VectorSubcoreMesh(core_axis_name="c", subcore_axis_name="s")
    per_tile = NCH * K

    def body(nf_hbm, src_hbm, dst_hbm, cnt_hbm, z_hbm, out_hbm,
             nf_loc, acc_loc, cnt_loc, dpl, deg_v, deg_loc,
             idx_s0, idx_s1, idx_s2, idx_s3,
             idx_d0, idx_d1, idx_d2, idx_d3,
             buf_a0, buf_a1, buf_b0, buf_b1,
             sem_i0, sem_i1, sem_i2, sem_i3,
             sem_ga0, sem_ga1, sem_gb0, sem_gb1,
             sem_sa0, sem_sa1, sem_sb0, sem_sb1,
             nf_sh, acc_sh, deg_part):
        c = lax.axis_index("c")
        s = lax.axis_index("s")
        r0 = s * ROWS
        idx_s = [idx_s0, idx_s1, idx_s2, idx_s3]
        idx_d = [idx_d0, idx_d1, idx_d2, idx_d3]
        buf_a = [buf_a0, buf_a1]
        buf_b = [buf_b0, buf_b1]
        sem_i = [sem_i0, sem_i1, sem_i2, sem_i3]
        sem_ga = [sem_ga0, sem_ga1]
        sem_gb = [sem_gb0, sem_gb1]
        sem_sa = [sem_sa0, sem_sa1]
        sem_sb = [sem_sb0, sem_sb1]
        zero16 = jnp.zeros((16,), jnp.float32)
        ones16 = jnp.full((16,), 1.0, jnp.float32)

        ebase = s * per_tile

        pltpu.sync_copy(z_hbm, acc_loc)
        for cc in range(ROWS // RSUB):
            rc = r0 + cc * RSUB
            pltpu.sync_copy(nf_hbm.at[c, pl.ds(rc, RSUB), :], nf_loc)
            pltpu.sync_copy(nf_loc, nf_sh.at[pl.ds(rc, RSUB)])
            pltpu.sync_copy(acc_loc, acc_sh.at[pl.ds(rc, RSUB)])

        def _zdeg(i, carry):
            deg_v[pl.ds(i * 16, 16)] = zero16
            return carry
        lax.fori_loop(0, N_PAD // 16, _zdeg, 0)

        plsc.subcore_barrier()

        def stage_idx(k, kk):
            j = kk % NIDX
            off = ebase + k * K
            pltpu.async_copy(src_hbm.at[pl.ds(off, K)], idx_s[j], sem_i[j])
            pltpu.async_copy(dst_hbm.at[pl.ds(off, K)], idx_d[j], sem_i[j])

        def wait_idx(k, kk):
            j = kk % NIDX
            off = ebase + k * K
            pltpu.make_async_copy(src_hbm.at[pl.ds(off, K)], idx_s[j], sem_i[j]).wait()
            pltpu.make_async_copy(dst_hbm.at[pl.ds(off, K)], idx_d[j], sem_i[j]).wait()

        def issue_gathers(kk):
            j = kk % NIDX
            b = kk % NBUF
            pltpu.async_copy(nf_sh.at[idx_d[j]], buf_b[b], sem_gb[b])
            pltpu.async_copy(nf_sh.at[idx_s[j]], buf_a[b], sem_ga[b])

        def wait_gathers(kk):
            j = kk % NIDX
            b = kk % NBUF
            pltpu.make_async_copy(nf_sh.at[idx_d[j]], buf_b[b], sem_gb[b]).wait()
            pltpu.make_async_copy(nf_sh.at[idx_s[j]], buf_a[b], sem_ga[b]).wait()

        def issue_scatters(kk):
            j = kk % NIDX
            b = kk % NBUF
            p = kk % 2
            pltpu.async_copy(buf_b[b], acc_sh.at[idx_s[j]], sem_sb[p], add=True)
            pltpu.async_copy(buf_a[b], acc_sh.at[idx_d[j]], sem_sa[p], add=True)

        def wait_scatters(kk):
            b = kk % NBUF
            p = kk % 2
            pltpu.make_async_copy(buf_b[b], acc_sh.at[idx_s[0]], sem_sb[p]).wait()
            pltpu.make_async_copy(buf_a[b], acc_sh.at[idx_d[0]], sem_sa[p]).wait()

        def chunk_body(k, kk, wait_prev, stage_next):
            if wait_prev:
                wait_scatters(kk - 2)
            if stage_next:
                stage_idx(k + 2, kk + 2)
            wait_idx(k, kk)
            j = kk % NIDX
            for jj in range(K // 16):
                plsc.addupdate_scatter(deg_v, [idx_s[j][pl.ds(jj * 16, 16)]], ones16)
                plsc.addupdate_scatter(deg_v, [idx_d[j][pl.ds(jj * 16, 16)]], ones16)
            issue_gathers(kk)
            wait_gathers(kk)
            issue_scatters(kk)

        stage_idx(0, 0)
        stage_idx(1, 1)
        chunk_body(0, 0, False, True)
        chunk_body(1, 1, False, True)
        chunk_body(2, 2, True, True)
        chunk_body(3, 3, True, True)

        def _iter(i, carry):
            kbase = i * NIDX
            for kk in range(NIDX):
                chunk_body(kbase + kk, kk, True, True)
            return carry
        lax.fori_loop(1, NCH // NIDX - 1, _iter, 0)

        for kk in range(NCH - 4, NCH):
            chunk_body(kk, kk % NIDX, True, stage_next=(kk + 2 < NCH))
        wait_scatters(NCH - 2)
        wait_scatters(NCH - 1)

        pltpu.sync_copy(deg_v, deg_part.at[s])
        plsc.subcore_barrier()
        for cc in range(ROWS // RSUB):
            pltpu.sync_copy(deg_part.at[:, pl.ds(r0 + cc * RSUB, RSUB)], dpl)

            def _gbody(g, carry):
                a = dpl[0, pl.ds(g * 16, 16)]
                for j in range(1, NT):
                    a = a + dpl[j, pl.ds(g * 16, 16)]
                deg_loc[pl.ds(cc * RSUB + g * 16, 16)] = a
                return carry
            lax.fori_loop(0, RSUB // 16, _gbody, 0)

        pltpu.sync_copy(cnt_hbm.at[pl.ds(r0, ROWS)], cnt_loc)
        for cc in range(ROWS // RSUB):
            rc = r0 + cc * RSUB
            pltpu.sync_copy(nf_hbm.at[c, pl.ds(rc, RSUB), :], nf_loc)
            pltpu.sync_copy(acc_sh.at[pl.ds(rc, RSUB)], acc_loc)

            def _egrp(g, carry):
                sc16 = 0.5 / cnt_loc[pl.ds(cc * RSUB + g * 16, 16)]
                dg16 = deg_loc[pl.ds(cc * RSUB + g * 16, 16)]
                for lane in range(16):
                    r = g * 16 + lane
                    dg = dg16[lane]
                    sc = sc16[lane]
                    for j in range(DH // 16):
                        sl = pl.ds(j * 16, 16)
                        acc_loc[r, sl] = (acc_loc[r, sl] + dg * nf_loc[r, sl]) * sc
                return carry
            lax.fori_loop(0, RSUB // 16, _egrp, 0)
            pltpu.sync_copy(acc_loc, out_hbm.at[c, pl.ds(rc, RSUB), :])

    return pl.kernel(
        body,
        out_type=jax.ShapeDtypeStruct((2, N_PAD, DH), jnp.float32),
        mesh=mesh,
        compiler_params=pltpu.CompilerParams(
            needs_layout_passes=False, use_tc_tiling_on_sc=False),
        scratch_types=(
            [
                pltpu.VMEM((RSUB, DH), jnp.float32),
                pltpu.VMEM((RSUB, DH), jnp.float32),
                pltpu.VMEM((ROWS,), jnp.float32),
                pltpu.VMEM((NT, RSUB), jnp.float32),
                pltpu.VMEM((N_PAD,), jnp.float32),
                pltpu.VMEM((ROWS,), jnp.float32),
            ]
            + [pltpu.VMEM((K,), jnp.int32) for _ in range(2 * NIDX)]
            + [pltpu.VMEM((K, DH), jnp.float32) for _ in range(2 * NBUF)]
            + [pltpu.SemaphoreType.DMA for _ in range(NIDX + 2 * NBUF + 4)]
            + [
                pltpu.VMEM_SHARED((N_PAD, DH), jnp.float32),
                pltpu.VMEM_SHARED((N_PAD, DH), jnp.float32),
                pltpu.VMEM_SHARED((NT, N_PAD), jnp.float32),
            ]
        ),
    )


def kernel(to_concat, node_features, edges, count):
    nf = node_features.astype(jnp.float32)
    src = edges[:, 0].astype(jnp.int32)
    dst = edges[:, 1].astype(jnp.int32)
    n_edges = src.shape[0]
    src = jnp.pad(src, (0, E_PAD - n_edges), constant_values=N_PAD - 1)
    dst = jnp.pad(dst, (0, E_PAD - n_edges), constant_values=N_PAD - 1)
    nf_pad = jnp.pad(nf, ((0, N_PAD - N_NODES), (0, 0)))
    nf_halves = jnp.transpose(nf_pad.reshape(N_PAD, 2, DH), (1, 0, 2))
    cnt = jnp.pad(count[:, 0].astype(jnp.float32), (0, N_PAD - N_NODES),
                  constant_values=1.0)
    zrows = jnp.zeros((RSUB, DH), jnp.float32)
    sm = _make_sc_kernel()(nf_halves, src, dst, cnt, zrows)
    sm = jnp.transpose(sm, (1, 0, 2)).reshape(N_PAD, D)
    return jnp.concatenate([to_concat, sm[:N_NODES]], axis=1)


# speedup vs baseline: 1.5014x; 1.1329x over previous
import jax
import jax.numpy as jnp
from jax import lax
from jax.experimental import pallas as pl
from jax.experimental.pallas import tpu as pltpu
from jax.experimental.pallas import tpu_sc as plsc

N_NODES = 10000
N_PAD = 10240
D = 128
DH = 64
K = 64
NT = 16
ROWS = N_PAD // NT
RSUB = 64
NCH = 320
E_PAD = NT * NCH * K
NIDX = 4
NBUF = 2


def _make_sc_kernel():
    mesh = plsc.VectorSubcoreMesh(core_axis_name="c", subcore_axis_name="s")
    per_tile = NCH * K

    def body(nf_hbm, src_hbm, dst_hbm, cnt_hbm, out_hbm,
             nf_loc, acc_loc, dpl, deg_v, deg_loc, cnt_loc,
             idx_s0, idx_s1, idx_s2, idx_s3,
             idx_d0, idx_d1, idx_d2, idx_d3,
             buf_a0, buf_a1, buf_b0, buf_b1,
             sem_i0, sem_i1, sem_i2, sem_i3,
             sem_ga0, sem_ga1, sem_gb0, sem_gb1,
             sem_sa0, sem_sa1, sem_sb0, sem_sb1,
             nf_sh, acc_sh, deg_part):
        c = lax.axis_index("c")
        s = lax.axis_index("s")
        r0 = s * ROWS
        zero16 = jnp.zeros((16,), jnp.float32)
        ones16 = jnp.full((16,), 1.0, jnp.float32)
        idx_s = [idx_s0, idx_s1, idx_s2, idx_s3]
        idx_d = [idx_d0, idx_d1, idx_d2, idx_d3]
        buf_a = [buf_a0, buf_a1]
        buf_b = [buf_b0, buf_b1]
        sem_i = [sem_i0, sem_i1, sem_i2, sem_i3]
        sem_ga = [sem_ga0, sem_ga1]
        sem_gb = [sem_gb0, sem_gb1]
        sem_sa = [sem_sa0, sem_sa1]
        sem_sb = [sem_sb0, sem_sb1]

        ebase = s * per_tile

        def _zrow(r, carry):
            for j in range(DH // 16):
                acc_loc[r, pl.ds(j * 16, 16)] = zero16
            return carry
        lax.fori_loop(0, RSUB, _zrow, 0)
        for cc in range(ROWS // RSUB):
            pltpu.sync_copy(acc_loc, acc_sh.at[pl.ds(r0 + cc * RSUB, RSUB)])

        for cc in range(ROWS // RSUB):
            rc = r0 + cc * RSUB
            pltpu.sync_copy(nf_hbm.at[c, pl.ds(rc, RSUB), :], nf_loc)
            pltpu.sync_copy(nf_loc, nf_sh.at[pl.ds(rc, RSUB)])

        def _zdeg(i, carry):
            deg_v[pl.ds(i * 16, 16)] = zero16
            return carry
        lax.fori_loop(0, N_PAD // 16, _zdeg, 0)

        plsc.subcore_barrier()

        def stage_idx(k, s4):
            off = ebase + k * K
            pltpu.async_copy(src_hbm.at[pl.ds(off, K)], idx_s[s4], sem_i[s4])
            pltpu.async_copy(dst_hbm.at[pl.ds(off, K)], idx_d[s4], sem_i[s4])

        def wait_idx(k, s4):
            off = ebase + k * K
            pltpu.make_async_copy(src_hbm.at[pl.ds(off, K)], idx_s[s4], sem_i[s4]).wait()
            pltpu.make_async_copy(dst_hbm.at[pl.ds(off, K)], idx_d[s4], sem_i[s4]).wait()

        def wait_scatters(s2):
            pltpu.make_async_copy(buf_b[s2], acc_sh.at[idx_s[0]], sem_sb[s2]).wait()
            pltpu.make_async_copy(buf_a[s2], acc_sh.at[idx_d[0]], sem_sa[s2]).wait()

        def chunk_body(k, kk, wait_prev, stage_next):
            s2 = kk % NBUF
            s4 = kk % NIDX
            if wait_prev:
                wait_scatters(s2)
            if stage_next:
                stage_idx(k + 2, (kk + 2) % NIDX)
            wait_idx(k, s4)
            for j in range(K // 16):
                plsc.addupdate_scatter(deg_v, [idx_s[s4][pl.ds(j * 16, 16)]], ones16)
                plsc.addupdate_scatter(deg_v, [idx_d[s4][pl.ds(j * 16, 16)]], ones16)
            pltpu.async_copy(nf_sh.at[idx_d[s4]], buf_b[s2], sem_gb[s2])
            pltpu.async_copy(nf_sh.at[idx_s[s4]], buf_a[s2], sem_ga[s2])
            pltpu.make_async_copy(nf_sh.at[idx_d[s4]], buf_b[s2], sem_gb[s2]).wait()
            pltpu.async_copy(buf_b[s2], acc_sh.at[idx_s[s4]], sem_sb[s2], add=True)
            pltpu.make_async_copy(nf_sh.at[idx_s[s4]], buf_a[s2], sem_ga[s2]).wait()
            pltpu.async_copy(buf_a[s2], acc_sh.at[idx_d[s4]], sem_sa[s2], add=True)

        for kk in range(NIDX):
            stage_idx(kk, kk)
        chunk_body(0, 0, wait_prev=False, stage_next=False)
        chunk_body(1, 1, wait_prev=False, stage_next=False)
        chunk_body(2, 2, wait_prev=True, stage_next=True)
        chunk_body(3, 3, wait_prev=True, stage_next=True)

        def _iter(i, carry):
            kbase = i * NIDX
            for kk in range(NIDX):
                chunk_body(kbase + kk, kk, wait_prev=True, stage_next=True)
            return carry
        lax.fori_loop(1, NCH // NIDX - 1, _iter, 0)

        chunk_body(NCH - 4, 0, wait_prev=True, stage_next=True)
        chunk_body(NCH - 3, 1, wait_prev=True, stage_next=True)
        chunk_body(NCH - 2, 2, wait_prev=True, stage_next=False)
        chunk_body(NCH - 1, 3, wait_prev=True, stage_next=False)
        for s2 in range(NBUF):
            wait_scatters(s2)

        pltpu.sync_copy(deg_v, deg_part.at[s])
        plsc.subcore_barrier()
        for cc in range(ROWS // RSUB):
            pltpu.sync_copy(deg_part.at[:, pl.ds(r0 + cc * RSUB, RSUB)], dpl)

            def _gbody(g, carry):
                a = dpl[0, pl.ds(g * 16, 16)]
                for j in range(1, NT):
                    a = a + dpl[j, pl.ds(g * 16, 16)]
                deg_loc[pl.ds(cc * RSUB + g * 16, 16)] = a
                return carry
            lax.fori_loop(0, RSUB // 16, _gbody, 0)

        pltpu.sync_copy(cnt_hbm.at[pl.ds(r0, ROWS)], cnt_loc)
        for cc in range(ROWS // RSUB):
            rc = r0 + cc * RSUB
            pltpu.sync_copy(nf_hbm.at[c, pl.ds(rc, RSUB), :], nf_loc)
            pltpu.sync_copy(acc_sh.at[pl.ds(rc, RSUB)], acc_loc)

            def _egrp(g, carry):
                dg16 = deg_loc[pl.ds(cc * RSUB + g * 16, 16)]
                sc16 = 0.5 / cnt_loc[pl.ds(cc * RSUB + g * 16, 16)]
                for lane in range(16):
                    r = g * 16 + lane
                    dg = dg16[lane]
                    sc = sc16[lane]
                    for j in range(DH // 16):
                        sl = pl.ds(j * 16, 16)
                        acc_loc[r, sl] = (acc_loc[r, sl] + dg * nf_loc[r, sl]) * sc
                return carry
            lax.fori_loop(0, RSUB // 16, _egrp, 0)
            pltpu.sync_copy(acc_loc, out_hbm.at[c, pl.ds(rc, RSUB), :])

    return pl.kernel(
        body,
        out_type=jax.ShapeDtypeStruct((2, N_PAD, DH), jnp.float32),
        mesh=mesh,
        compiler_params=pltpu.CompilerParams(
            needs_layout_passes=False, use_tc_tiling_on_sc=False),
        scratch_types=(
            [
                pltpu.VMEM((RSUB, DH), jnp.float32),
                pltpu.VMEM((RSUB, DH), jnp.float32),
                pltpu.VMEM((NT, RSUB), jnp.float32),
                pltpu.VMEM((N_PAD,), jnp.float32),
                pltpu.VMEM((ROWS,), jnp.float32),
                pltpu.VMEM((ROWS,), jnp.float32),
            ]
            + [pltpu.VMEM((K,), jnp.int32) for _ in range(2 * NIDX)]
            + [pltpu.VMEM((K, DH), jnp.float32) for _ in range(2 * NBUF)]
            + [pltpu.SemaphoreType.DMA for _ in range(NIDX + 4 * NBUF)]
            + [
                pltpu.VMEM_SHARED((N_PAD, DH), jnp.float32),
                pltpu.VMEM_SHARED((N_PAD, DH), jnp.float32),
                pltpu.VMEM_SHARED((NT, N_PAD), jnp.float32),
            ]
        ),
    )


def kernel(to_concat, node_features, edges, count):
    nf = node_features.astype(jnp.float32)
    src = edges[:, 0].astype(jnp.int32)
    dst = edges[:, 1].astype(jnp.int32)
    n_edges = src.shape[0]
    src = jnp.pad(src, (0, E_PAD - n_edges), constant_values=N_PAD - 1)
    dst = jnp.pad(dst, (0, E_PAD - n_edges), constant_values=N_PAD - 1)
    nf_pad = jnp.pad(nf, ((0, N_PAD - N_NODES), (0, 0)))
    nf_halves = jnp.transpose(nf_pad.reshape(N_PAD, 2, DH), (1, 0, 2))
    cnt = jnp.pad(count[:, 0].astype(jnp.float32), (0, N_PAD - N_NODES),
                  constant_values=1.0)
    sm = _make_sc_kernel()(nf_halves, src, dst, cnt)
    sm = jnp.transpose(sm, (1, 0, 2)).reshape(N_PAD, D)
    return jnp.concatenate([to_concat, sm[:N_NODES]], axis=1)
